# baseline (speedup 1.0000x reference)
import jax
import jax.numpy as jnp
from jax import lax
from jax.experimental import pallas as pl
from jax.experimental.pallas import tpu as pltpu
from jax.experimental.pallas import tpu_sc as plsc

B = 4
H = W = 192
C = 96
OH = OW = 384
HW = H * W
OHW = OH * OW
CG = 4
NG = C // CG
NPAIR = NG // 2
NC = 2
NS = 16
ROWS_PER_TILE = HW // NS
NVEC = ROWS_PER_TILE // 2
NEL = ROWS_PER_TILE * 8
ACC_WORDS = OHW * CG
STRIPE = ACC_WORDS // NS
BRT = 1024


def _scatter_kernel(b_base, upd_hbm, msk_hbm, zer_hbm, perm_hbm, valb_v,
                    mskb_v, vals_a, idx_a, idx_b, acc_sh):
  s = lax.axis_index("s")

  iot = lax.iota(jnp.int32, 16)
  f_off = (iot & 3) * OHW
  m_a = (iot & 4) == 0
  m_b = (iot & 4) != 0

  w0 = s * STRIPE

  def do_pair(b, bb, q):
    ch0 = q * 8

    q0 = s * NVEC
    pltpu.sync_copy(msk_hbm.at[b, pl.ds(q0, NVEC), 0, pl.ds(ch0, 8)],
                    mskb_v.at[:, pl.ds(0, 8)])
    pltpu.sync_copy(msk_hbm.at[b, pl.ds(q0, NVEC), 1, pl.ds(ch0, 8)],
                    mskb_v.at[:, pl.ds(8, 8)])
    pltpu.sync_copy(upd_hbm.at[b, pl.ds(q0, NVEC), 0, pl.ds(ch0, 8)],
                    valb_v.at[:, pl.ds(0, 8)])
    pltpu.sync_copy(upd_hbm.at[b, pl.ds(q0, NVEC), 1, pl.ds(ch0, 8)],
                    valb_v.at[:, pl.ds(8, 8)])

    def _decode(i, _):
      m = mskb_v[i]
      t = lax.shift_right_logical(m, 12)
      y = lax.shift_right_logical(t * 7282, 16)
      r = m - ((y << 15) + (y << 12))
      t2 = lax.shift_right_logical(r, 5)
      x = lax.shift_right_logical(t2 * 21846, 16)
      loc = f_off + (y << 8) + (y << 7) + x
      neg1 = jnp.full((16,), -1, jnp.int32)
      o = i * 16
      idx_a[pl.ds(o, 16)] = jnp.where(m_a, loc, neg1)
      idx_b[pl.ds(o, 16)] = jnp.where(m_b, loc, neg1)
      vals_a[pl.ds(o, 16)] = valb_v[i]
      return _
    lax.fori_loop(0, NVEC, _decode, None)

    for half, idx_v in enumerate((idx_a, idx_b)):
      slab = (bb * NG + 2 * q + half) * ACC_WORDS

      pltpu.sync_copy(zer_hbm.at[pl.ds(w0, STRIPE)],
                      acc_sh.at[pl.ds(w0, STRIPE)])

      plsc.subcore_barrier()

      pltpu.sync_copy(
          vals_a,
          acc_sh.at[plsc.Indices(idx_v, ignored_value=-1)],
          add=True)

      plsc.subcore_barrier()
      plsc.subcore_barrier()

      pltpu.sync_copy(acc_sh.at[pl.ds(w0, STRIPE)],
                      perm_hbm.at[pl.ds(slab + w0, STRIPE)])

  for bb in range(B // NC):
    b = b_base + bb
    def _qloop(q, _):
      do_pair(b, bb, q)
      return _
    lax.fori_loop(0, NPAIR, _qloop, None)


def _interleave_body(p0_ref, p1_ref, out_ref):
  b = pl.program_id(0)
  x = jnp.where(b < B // 2, p0_ref[...], p1_ref[...])
  out_ref[...] = jnp.transpose(x, (0, 2, 1))


@jax.jit
def kernel(updates, mask):
  msk4 = mask.astype(jnp.int32).reshape(B, HW // 2, 2, C)
  upd4 = updates.reshape(B, HW // 2, 2, C)

  mesh = plsc.VectorSubcoreMesh(core_axis_name="c", subcore_axis_name="s",
                                num_cores=1)
  params = pltpu.CompilerParams(use_tc_tiling_on_sc=False)
  import functools
  half_words = (B // 2) * C * OHW
  scratch = [
      pltpu.VMEM((NVEC, 16), jnp.float32),
      pltpu.VMEM((NVEC, 16), jnp.int32),
      pltpu.VMEM((NEL,), jnp.float32),
      pltpu.VMEM((NEL,), jnp.int32),
      pltpu.VMEM((NEL,), jnp.int32),
      pltpu.VMEM_SHARED((ACC_WORDS,), jnp.float32),
  ]
  zer1 = jnp.zeros((ACC_WORDS,), jnp.float32)
  perms = []
  for b_base in (0, B // 2):
    fn = pl.kernel(
        functools.partial(_scatter_kernel, b_base),
        out_type=jax.ShapeDtypeStruct((half_words,), jnp.float32),
        mesh=mesh,
        compiler_params=params,
        scratch_types=scratch,
        name=f"unpool_scatter_b{b_base}",
    )
    perms.append(fn(upd4, msk4, zer1))

  out = pl.pallas_call(
      _interleave_body,
      out_shape=jax.ShapeDtypeStruct((B, OHW, C), jnp.float32),
      grid=(B, OHW // BRT),
      in_specs=[
          pl.BlockSpec((1, C, BRT), lambda b, r: (jnp.minimum(b, 1), 0, r)),
          pl.BlockSpec((1, C, BRT),
                       lambda b, r: (jnp.maximum(b, 2) - 2, 0, r)),
      ],
      out_specs=pl.BlockSpec((1, BRT, C), lambda b, r: (b, r, 0)),
  )(perms[0].reshape(B // 2, C, OHW), perms[1].reshape(B // 2, C, OHW))
  return out.reshape(B, OH, OW, C)


# speedup vs baseline: 1.4910x; 1.4910x over previous
import jax
import jax.numpy as jnp
from jax import lax
from jax.experimental import pallas as pl
from jax.experimental.pallas import tpu as pltpu
from jax.experimental.pallas import tpu_sc as plsc

B = 4
H = W = 192
C = 96
OH = OW = 384
HW = H * W
OHW = OH * OW
CG = 4
NG = C // CG
NPAIR = NG // 2
NC = 2
NS = 16
NSUB = 2
ROWS_PER_TILE = HW // NS
NVEC = ROWS_PER_TILE // 2
SVEC = NVEC // NSUB
NELS = SVEC * 16
ACC_WORDS = OHW * CG
ACC2 = 2 * ACC_WORDS
STRIPE = ACC2 // NS
BRT = 1024


def _scatter_kernel(upd_hbm, msk_hbm, zer_hbm, perm_hbm, valb_v, mskb_v,
                    vals_v, idx_v, acc_sh):
  c = lax.axis_index("c")
  s = lax.axis_index("s")

  iot = lax.iota(jnp.int32, 16)
  f_off = (iot & 3) * OHW + (lax.shift_right_logical(iot, 2) & 1) * ACC_WORDS

  w2 = s * STRIPE

  def do_pair(b, bb, q):
    ch0 = q * 8

    pltpu.sync_copy(zer_hbm.at[pl.ds(w2, STRIPE)],
                    acc_sh.at[pl.ds(w2, STRIPE)])

    for sub in range(NSUB):
      q0 = s * NVEC + sub * SVEC
      pltpu.sync_copy(msk_hbm.at[b, pl.ds(q0, SVEC), 0, pl.ds(ch0, 8)],
                      mskb_v.at[:, pl.ds(0, 8)])
      pltpu.sync_copy(msk_hbm.at[b, pl.ds(q0, SVEC), 1, pl.ds(ch0, 8)],
                      mskb_v.at[:, pl.ds(8, 8)])
      pltpu.sync_copy(upd_hbm.at[b, pl.ds(q0, SVEC), 0, pl.ds(ch0, 8)],
                      valb_v.at[:, pl.ds(0, 8)])
      pltpu.sync_copy(upd_hbm.at[b, pl.ds(q0, SVEC), 1, pl.ds(ch0, 8)],
                      valb_v.at[:, pl.ds(8, 8)])

      def _decode(i, _):
        m = mskb_v[i]
        t = lax.shift_right_logical(m, 12)
        y = lax.shift_right_logical(t * 7282, 16)
        r = m - ((y << 15) + (y << 12))
        t2 = lax.shift_right_logical(r, 5)
        x = lax.shift_right_logical(t2 * 21846, 16)
        idx_v[pl.ds(i * 16, 16)] = f_off + (y << 8) + (y << 7) + x
        vals_v[pl.ds(i * 16, 16)] = valb_v[i]
        return _
      lax.fori_loop(0, SVEC, _decode, None)

      if sub == 0:
        plsc.subcore_barrier()

      pltpu.sync_copy(vals_v, acc_sh.at[idx_v], add=True)

    plsc.subcore_barrier()
    plsc.subcore_barrier()

    pair_base = ((b * NG + 2 * q) * ACC_WORDS)
    pltpu.sync_copy(acc_sh.at[pl.ds(w2, STRIPE)],
                    perm_hbm.at[pl.ds(pair_base + w2, STRIPE)])

  for bb in range(B // NC):
    b = c * (B // NC) + bb
    def _qloop(q, _):
      do_pair(b, bb, q)
      return _
    lax.fori_loop(0, NPAIR, _qloop, None)


def _interleave_body(perm_ref, out_ref):
  out_ref[...] = jnp.transpose(perm_ref[...], (0, 2, 1))


@jax.jit
def kernel(updates, mask):
  msk4 = mask.astype(jnp.int32).reshape(B, HW // 2, 2, C)
  upd4 = updates.reshape(B, HW // 2, 2, C)

  mesh = plsc.VectorSubcoreMesh(core_axis_name="c", subcore_axis_name="s")
  params = pltpu.CompilerParams(use_tc_tiling_on_sc=False)
  scatter_fn = pl.kernel(
      _scatter_kernel,
      out_type=jax.ShapeDtypeStruct((B * C * OHW,), jnp.float32),
      mesh=mesh,
      compiler_params=params,
      scratch_types=[
          pltpu.VMEM((SVEC, 16), jnp.float32),
          pltpu.VMEM((SVEC, 16), jnp.int32),
          pltpu.VMEM((NELS,), jnp.float32),
          pltpu.VMEM((NELS,), jnp.int32),
          pltpu.VMEM_SHARED((ACC2,), jnp.float32),
      ],
  )
  zer1 = jnp.zeros((ACC2,), jnp.float32)
  perm = scatter_fn(upd4, msk4, zer1)

  out = pl.pallas_call(
      _interleave_body,
      out_shape=jax.ShapeDtypeStruct((B, OHW, C), jnp.float32),
      grid=(B, OHW // BRT),
      in_specs=[pl.BlockSpec((1, C, BRT), lambda b, r: (b, 0, r))],
      out_specs=pl.BlockSpec((1, BRT, C), lambda b, r: (b, r, 0)),
  )(perm.reshape(B, C, OHW))
  return out.reshape(B, OH, OW, C)


# speedup vs baseline: 1.5881x; 1.0651x over previous
import jax
import jax.numpy as jnp
from jax import lax
from jax.experimental import pallas as pl
from jax.experimental.pallas import tpu as pltpu
from jax.experimental.pallas import tpu_sc as plsc

B = 4
H = W = 192
C = 96
OH = OW = 384
HW = H * W
OHW = OH * OW
CG = 4
NG = C // CG
NPAIR = NG // 2
NC = 2
NS = 16
NSUB = 2
ROWS_PER_TILE = HW // NS
NVEC = ROWS_PER_TILE // 2
SVEC = NVEC // NSUB
NELS = SVEC * 16
ACC_WORDS = OHW * CG
ACC2 = 2 * ACC_WORDS
STRIPE = ACC2 // NS
BRT = 1024


def _scatter_kernel(upd_hbm, msk_hbm, zer_hbm, perm_hbm, valb0_v, mskb0_v,
                    valb1_v, mskb1_v, vals_v, idx_v, acc_sh, gsem, zsem):
  c = lax.axis_index("c")
  s = lax.axis_index("s")

  iot = lax.iota(jnp.int32, 16)
  f_off = (iot & 3) * OHW + (lax.shift_right_logical(iot, 2) & 1) * ACC_WORDS

  w2 = s * STRIPE

  def do_pair(b, bb, q):
    ch0 = q * 8

    zdesc = pltpu.async_copy(zer_hbm.at[pl.ds(w2, STRIPE)],
                             acc_sh.at[pl.ds(w2, STRIPE)], zsem)

    q0 = s * NVEC
    descs = []
    for sub, (mskb_v, valb_v) in enumerate(((mskb0_v, valb0_v),
                                            (mskb1_v, valb1_v))):
      qs = q0 + sub * SVEC
      for src, dst in (
          (msk_hbm.at[b, pl.ds(qs, SVEC), 0, pl.ds(ch0, 8)],
           mskb_v.at[:, pl.ds(0, 8)]),
          (msk_hbm.at[b, pl.ds(qs, SVEC), 1, pl.ds(ch0, 8)],
           mskb_v.at[:, pl.ds(8, 8)]),
          (upd_hbm.at[b, pl.ds(qs, SVEC), 0, pl.ds(ch0, 8)],
           valb_v.at[:, pl.ds(0, 8)]),
          (upd_hbm.at[b, pl.ds(qs, SVEC), 1, pl.ds(ch0, 8)],
           valb_v.at[:, pl.ds(8, 8)])):
        descs.append(pltpu.async_copy(src, dst, gsem))

    for sub, (mskb_v, valb_v) in enumerate(((mskb0_v, valb0_v),
                                            (mskb1_v, valb1_v))):
      for d in descs[4 * sub:4 * sub + 4]:
        d.wait()

      def _decode(i, _):
        m = mskb_v[i]
        t = lax.shift_right_logical(m, 12)
        y = lax.shift_right_logical(t * 7282, 16)
        r = m - ((y << 15) + (y << 12))
        t2 = lax.shift_right_logical(r, 5)
        x = lax.shift_right_logical(t2 * 21846, 16)
        idx_v[pl.ds(i * 16, 16)] = f_off + (y << 8) + (y << 7) + x
        vals_v[pl.ds(i * 16, 16)] = valb_v[i]
        return _
      lax.fori_loop(0, SVEC, _decode, None)

      if sub == 0:
        zdesc.wait()
        plsc.subcore_barrier()

      pltpu.sync_copy(vals_v, acc_sh.at[idx_v], add=True)

    plsc.subcore_barrier()
    plsc.subcore_barrier()

    pair_base = ((b * NG + 2 * q) * ACC_WORDS)
    pltpu.sync_copy(acc_sh.at[pl.ds(w2, STRIPE)],
                    perm_hbm.at[pl.ds(pair_base + w2, STRIPE)])

  for bb in range(B // NC):
    b = c * (B // NC) + bb
    def _qloop(q, _):
      do_pair(b, bb, q)
      return _
    lax.fori_loop(0, NPAIR, _qloop, None)


def _interleave_body(perm_ref, out_ref):
  out_ref[...] = jnp.transpose(perm_ref[...], (0, 2, 1))


@jax.jit
def kernel(updates, mask):
  msk4 = mask.astype(jnp.int32).reshape(B, HW // 2, 2, C)
  upd4 = updates.reshape(B, HW // 2, 2, C)

  mesh = plsc.VectorSubcoreMesh(core_axis_name="c", subcore_axis_name="s")
  params = pltpu.CompilerParams(use_tc_tiling_on_sc=False)
  scatter_fn = pl.kernel(
      _scatter_kernel,
      out_type=jax.ShapeDtypeStruct((B * C * OHW,), jnp.float32),
      mesh=mesh,
      compiler_params=params,
      scratch_types=[
          pltpu.VMEM((SVEC, 16), jnp.float32),
          pltpu.VMEM((SVEC, 16), jnp.int32),
          pltpu.VMEM((SVEC, 16), jnp.float32),
          pltpu.VMEM((SVEC, 16), jnp.int32),
          pltpu.VMEM((NELS,), jnp.float32),
          pltpu.VMEM((NELS,), jnp.int32),
          pltpu.VMEM_SHARED((ACC2,), jnp.float32),
          pltpu.SemaphoreType.DMA,
          pltpu.SemaphoreType.DMA,
      ],
  )
  zer1 = jnp.zeros((ACC2,), jnp.float32)
  perm = scatter_fn(upd4, msk4, zer1)

  out = pl.pallas_call(
      _interleave_body,
      out_shape=jax.ShapeDtypeStruct((B, OHW, C), jnp.float32),
      grid=(B, OHW // BRT),
      in_specs=[pl.BlockSpec((1, C, BRT), lambda b, r: (b, 0, r))],
      out_specs=pl.BlockSpec((1, BRT, C), lambda b, r: (b, r, 0)),
  )(perm.reshape(B, C, OHW))
  return out.reshape(B, OH, OW, C)
